# initial kernel scaffold (unmeasured)
import jax
import jax.numpy as jnp
from jax import lax
from jax.experimental import pallas as pl
from jax.experimental.pallas import tpu as pltpu


def kernel(
    x,
):
    def body(*refs):
        pass

    out_shape = jax.ShapeDtypeStruct(..., jnp.float32)
    return pl.pallas_call(body, out_shape=out_shape)(...)


# baseline (device time: 79658 ns/iter reference)
import jax
import jax.numpy as jnp
from jax import lax
from jax.experimental import pallas as pl
from jax.experimental.pallas import tpu as pltpu

N_Z = 4


def kernel(x):
    m_per, n = x.shape
    n_per = n // N_Z

    def body(x_ref, out_ref, send_sems, recv_sems):
        my_x = lax.axis_index("x")
        my_y = lax.axis_index("y")
        my_z = lax.axis_index("z")

        barrier_sem = pltpu.get_barrier_semaphore()
        for d in range(1, N_Z):
            peer = (my_z + d) % N_Z
            pl.semaphore_signal(
                barrier_sem, inc=1,
                device_id=(my_x, my_y, peer),
                device_id_type=pl.DeviceIdType.MESH,
            )
        pl.semaphore_wait(barrier_sem, N_Z - 1)

        rdmas = []
        for d in range(1, N_Z):
            tgt = (my_z + d) % N_Z
            rdma = pltpu.make_async_remote_copy(
                src_ref=x_ref.at[:, pl.ds(tgt * n_per, n_per)],
                dst_ref=out_ref.at[pl.ds(my_z * m_per, m_per), :],
                send_sem=send_sems.at[d - 1],
                recv_sem=recv_sems.at[d - 1],
                device_id=(my_x, my_y, tgt),
                device_id_type=pl.DeviceIdType.MESH,
            )
            rdma.start()
            rdmas.append(rdma)

        out_ref[pl.ds(my_z * m_per, m_per), :] = x_ref[
            :, pl.ds(my_z * n_per, n_per)
        ]

        for rdma in rdmas:
            rdma.wait()

    return pl.pallas_call(
        body,
        out_shape=jax.ShapeDtypeStruct((N_Z * m_per, n_per), x.dtype),
        in_specs=[pl.BlockSpec(memory_space=pltpu.VMEM)],
        out_specs=pl.BlockSpec(memory_space=pltpu.VMEM),
        scratch_shapes=[
            pltpu.SemaphoreType.DMA((N_Z - 1,)),
            pltpu.SemaphoreType.DMA((N_Z - 1,)),
        ],
        compiler_params=pltpu.CompilerParams(collective_id=0),
    )(x)


# device time: 66417 ns/iter; 1.1994x vs baseline; 1.1994x over previous
import jax
import jax.numpy as jnp
from jax import lax
from jax.experimental import pallas as pl
from jax.experimental.pallas import tpu as pltpu

N_Z = 4
N_COL = 4


def kernel(x):
    m_per, n = x.shape
    n_per = n // N_Z
    mq = m_per // N_COL

    def body(x_ref, out_ref, z_send, z_recv, xy_send, xy_recv):
        X = lax.axis_index("x")
        Y = lax.axis_index("y")
        k = lax.axis_index("z")
        c = 2 * X + Y

        xnbr = (1 - X, Y, k)
        ynbr = (X, 1 - Y, k)
        diag = (1 - X, 1 - Y, k)

        barrier_sem = pltpu.get_barrier_semaphore()
        peers = [(X, Y, (k + d) % N_Z) for d in range(1, N_Z)]
        peers += [xnbr, ynbr, diag]
        for peer in peers:
            pl.semaphore_signal(
                barrier_sem, inc=1,
                device_id=peer, device_id_type=pl.DeviceIdType.MESH,
            )
        pl.semaphore_wait(barrier_sem, len(peers))

        z_rdmas = {}
        for d in (3, 2, 1):
            kk = (k + d) % N_Z
            rdma = pltpu.make_async_remote_copy(
                src_ref=x_ref.at[pl.ds(c * mq, mq), pl.ds(kk * n_per, n_per)],
                dst_ref=out_ref.at[pl.ds(k * m_per + c * mq, mq), :],
                send_sem=z_send.at[d - 1],
                recv_sem=z_recv.at[d - 1],
                device_id=(X, Y, kk),
                device_id_type=pl.DeviceIdType.MESH,
            )
            rdma.start()
            z_rdmas[d] = rdma

        out_ref[pl.ds(k * m_per, m_per), :] = x_ref[
            :, pl.ds(k * n_per, n_per)
        ]

        xy_rdmas = []
        for j_idx, d in enumerate((1, 2, 3)):
            j = (k + d) % N_Z
            z_rdmas[d].wait_recv()
            row0 = j * m_per + c * mq
            for tgt, slot in ((diag, 2), (xnbr, 0), (ynbr, 1)):
                sidx = j_idx * 3 + slot
                rdma = pltpu.make_async_remote_copy(
                    src_ref=out_ref.at[pl.ds(row0, mq), :],
                    dst_ref=out_ref.at[pl.ds(row0, mq), :],
                    send_sem=xy_send.at[sidx],
                    recv_sem=xy_recv.at[sidx],
                    device_id=tgt,
                    device_id_type=pl.DeviceIdType.MESH,
                )
                rdma.start()
                xy_rdmas.append(rdma)

        for rdma in xy_rdmas:
            rdma.wait_recv()
        for d in (3, 2, 1):
            z_rdmas[d].wait_send()
        for rdma in xy_rdmas:
            rdma.wait_send()

    return pl.pallas_call(
        body,
        out_shape=jax.ShapeDtypeStruct((N_Z * m_per, n_per), x.dtype),
        in_specs=[pl.BlockSpec(memory_space=pltpu.VMEM)],
        out_specs=pl.BlockSpec(memory_space=pltpu.VMEM),
        scratch_shapes=[
            pltpu.SemaphoreType.DMA((N_Z - 1,)),
            pltpu.SemaphoreType.DMA((N_Z - 1,)),
            pltpu.SemaphoreType.DMA((3 * (N_Z - 1),)),
            pltpu.SemaphoreType.DMA((3 * (N_Z - 1),)),
        ],
        compiler_params=pltpu.CompilerParams(collective_id=0),
    )(x)
